# baseline (speedup 1.0000x reference)
import functools

import jax
import jax.numpy as jnp
from jax import lax
from jax.experimental import pallas as pl
from jax.experimental.pallas import tpu as pltpu
from jax.experimental.pallas import tpu_sc as plsc

_NUM_SC = 2
_NUM_TILES = 16
_CHUNK = 40
_ZROWS = 128


def _pad_rows(n_nodes):
    per_tile = -(-n_nodes // _NUM_TILES)
    per_tile = -(-per_tile // _ZROWS) * _ZROWS
    return per_tile


@functools.lru_cache(maxsize=None)
def _make_seg_sum(n_nodes, dim, n_edges):
    nw = _NUM_SC * _NUM_TILES
    per_tile = n_edges // nw
    assert per_tile * nw == n_edges and per_tile % _CHUNK == 0
    n_chunks = per_tile // _CHUNK
    rows_per_tile = _pad_rows(n_nodes)
    n_pad = rows_per_tile * _NUM_TILES
    zcopies = rows_per_tile // _ZROWS
    lanes = dim // 16

    mesh = plsc.VectorSubcoreMesh(core_axis_name="c", subcore_axis_name="s")

    _NBUF = 7
    assert n_chunks >= 3 * _NBUF
    _GRP = (n_chunks - _NBUF + 1) // _NBUF

    @functools.partial(
        pl.kernel,
        out_type=jax.ShapeDtypeStruct((_NUM_SC * n_pad, dim), jnp.float32),
        mesh=mesh,
        scratch_types=[
            pltpu.VMEM((per_tile,), jnp.int32),
            [pltpu.VMEM((_CHUNK,), jnp.int32)] * _NBUF,
            [pltpu.VMEM((_CHUNK, dim), jnp.float32)] * _NBUF,
            pltpu.VMEM_SHARED((n_pad, dim), jnp.float32),
            [pltpu.SemaphoreType.DMA] * _NBUF,
            [pltpu.SemaphoreType.DMA] * _NBUF,
            [pltpu.SemaphoreType.DMA] * _NBUF,
        ],
    )
    def seg_sum(x_hbm, ei_hbm, out_hbm, sidx, didxs, rows,
                acc, gsems, dsems, ssems):
        c = lax.axis_index("c")
        s = lax.axis_index("s")
        wid = c * _NUM_TILES + s
        base0 = wid * per_tile

        pltpu.sync_copy(ei_hbm.at[pl.ds(base0, per_tile)], sidx)

        def start_chunk(i, b):
            pltpu.async_copy(
                ei_hbm.at[pl.ds(n_edges + base0 + i * _CHUNK, _CHUNK)],
                didxs[b], dsems[b])
            pltpu.async_copy(
                x_hbm.at[sidx.at[pl.ds(i * _CHUNK, _CHUNK)]], rows[b],
                gsems[b])

        def wait_chunk(i, b):
            pltpu.make_async_copy(
                x_hbm.at[sidx.at[pl.ds(i * _CHUNK, _CHUNK)]], rows[b],
                gsems[b]).wait()
            pltpu.make_async_copy(
                ei_hbm.at[pl.ds(n_edges + base0 + i * _CHUNK, _CHUNK)],
                didxs[b], dsems[b]).wait()

        def wait_scatter(b):
            pltpu.make_async_copy(rows[b], acc.at[didxs[b]], ssems[b]).wait()

        def stage(i, b, prefetch, wait_prev):
            wait_chunk(i, b)
            pltpu.async_copy(rows[b], acc.at[didxs[b]], ssems[b], add=True)
            nb = (b + _NBUF - 1) % _NBUF
            if wait_prev:
                wait_scatter(nb)
            if prefetch:
                start_chunk(i + _NBUF - 1, nb)

        for j in range(_NBUF - 1):
            start_chunk(j, j)

        def zrow(r, carry):
            for u in range(lanes):
                rows[_NBUF - 1][r, pl.ds(16 * u, 16)] = (
                    jnp.zeros((16,), jnp.float32))
            return carry

        lax.fori_loop(0, _CHUNK, zrow, 0)

        last = n_nodes - (_NUM_TILES - 1) * rows_per_tile
        assert 0 < last <= rows_per_tile and last % 8 == 0
        assert (n_pad - n_nodes) % _CHUNK == 0

        @pl.when(jnp.logical_and(c == 0, s < _NUM_TILES - 1))
        def _():
            pltpu.sync_copy(x_hbm.at[pl.ds(s * rows_per_tile, rows_per_tile)],
                            acc.at[pl.ds(s * rows_per_tile, rows_per_tile)])

        @pl.when(jnp.logical_and(c == 0, s == _NUM_TILES - 1))
        def _():
            base = (_NUM_TILES - 1) * rows_per_tile
            pltpu.sync_copy(x_hbm.at[pl.ds(base, last)],
                            acc.at[pl.ds(base, last)])
            for z in range((n_pad - n_nodes) // _CHUNK):
                pltpu.sync_copy(
                    rows[_NBUF - 1],
                    acc.at[pl.ds(n_nodes + z * _CHUNK, _CHUNK)])

        @pl.when(c == 1)
        def _():
            for z in range(rows_per_tile // _CHUNK):
                pltpu.sync_copy(
                    rows[_NBUF - 1],
                    acc.at[pl.ds(s * rows_per_tile + z * _CHUNK, _CHUNK)])

        plsc.subcore_barrier()
        stage(0, 0, True, False)
        for j in range(1, _NBUF):
            stage(j, j, True, True)

        def group(p, carry):
            i0 = _NBUF * p
            for u in range(_NBUF):
                stage(i0 + u, u, True, True)
            return carry

        lax.fori_loop(1, _GRP, group, 0)
        for i in range(_NBUF * _GRP, n_chunks):
            stage(i, i % _NBUF, i + _NBUF - 1 < n_chunks, True)
        wait_scatter((n_chunks - 1) % _NBUF)

        plsc.subcore_barrier()
        pltpu.sync_copy(
            acc.at[pl.ds(s * rows_per_tile, rows_per_tile)],
            out_hbm.at[pl.ds(c * n_pad + s * rows_per_tile, rows_per_tile)])

    return seg_sum


def _gin_dense(n, p_ref, w1_ref, b1_ref, g_ref, bt_ref, w2_ref, b2_ref):
    n_pad = p_ref.shape[0] // 2
    h = p_ref[:n, :] + p_ref[n_pad:n_pad + n, :]
    t = (jnp.dot(h, w1_ref[...], preferred_element_type=jnp.float32)
         + b1_ref[...][None, :])
    m = jnp.mean(t, axis=0, keepdims=True)
    d = t - m
    v = jnp.mean(d * d, axis=0, keepdims=True)
    t = (g_ref[...][None, :] * d * lax.rsqrt(v + 1e-5)
         + bt_ref[...][None, :])
    t = jnp.maximum(t, 0.0)
    t = (jnp.dot(t, w2_ref[...], preferred_element_type=jnp.float32)
         + b2_ref[...][None, :])
    return jnp.maximum(t, 0.0)


def _dense_body(p_ref, w1_ref, b1_ref, g_ref, bt_ref, w2_ref, b2_ref,
                o_ref):
    o_ref[...] = _gin_dense(o_ref.shape[0], p_ref, w1_ref, b1_ref, g_ref,
                            bt_ref, w2_ref, b2_ref)


def _final_body(p_ref, batch_ref, w1_ref, b1_ref, g_ref, bt_ref,
                w2_ref, b2_ref, mw1_ref, mb1_ref, mw2_ref, mb2_ref, o_ref):
    n = batch_ref.shape[0]
    h = _gin_dense(n, p_ref, w1_ref, b1_ref, g_ref, bt_ref, w2_ref, b2_ref)
    n_graphs = o_ref.shape[0]
    onehot_t = (lax.broadcasted_iota(jnp.int32, (n_graphs, n), 0)
                == batch_ref[...][None, :]).astype(jnp.float32)
    pooled = jnp.dot(onehot_t, h, preferred_element_type=jnp.float32)
    u = jnp.maximum(
        jnp.dot(pooled, mw1_ref[...], preferred_element_type=jnp.float32)
        + mb1_ref[...][None, :], 0.0)
    o_ref[...] = (jnp.dot(u, mw2_ref[...], preferred_element_type=jnp.float32)
                  + mb2_ref[...][None, :])


def kernel(x, edge_index, batch, batch_size, c0_W1, c0_b1, c0_g, c0_bt, c0_W2,
           c0_b2, c1_W1, c1_b1, c1_g, c1_bt, c1_W2, c1_b2, m_W1, m_b1, m_W2,
           m_b2):
    n, d = x.shape
    e = edge_index.shape[1]
    hid = c0_W1.shape[1]
    out_d = m_W2.shape[1]
    n_graphs = batch_size if isinstance(batch_size, int) else 64

    ei_flat = edge_index.reshape(-1)

    seg_sum = _make_seg_sum(n, d, e)

    parts0 = seg_sum(x, ei_flat)
    h0 = pl.pallas_call(
        _dense_body,
        out_shape=jax.ShapeDtypeStruct((n, hid), jnp.float32),
    )(parts0, c0_W1, c0_b1, c0_g, c0_bt, c0_W2, c0_b2)

    parts1 = seg_sum(h0, ei_flat)
    out = pl.pallas_call(
        _final_body,
        out_shape=jax.ShapeDtypeStruct((n_graphs, out_d), jnp.float32),
    )(parts1, batch, c1_W1, c1_b1, c1_g, c1_bt, c1_W2,
      c1_b2, m_W1, m_b1, m_W2, m_b2)
    return out


# speedup vs baseline: 1.0019x; 1.0019x over previous
import functools

import jax
import jax.numpy as jnp
from jax import lax
from jax.experimental import pallas as pl
from jax.experimental.pallas import tpu as pltpu
from jax.experimental.pallas import tpu_sc as plsc

_NUM_SC = 2
_NUM_TILES = 16
_CHUNK = 40
_ZROWS = 128


def _pad_rows(n_nodes):
    per_tile = -(-n_nodes // _NUM_TILES)
    per_tile = -(-per_tile // _ZROWS) * _ZROWS
    return per_tile


@functools.lru_cache(maxsize=None)
def _make_seg_sum(n_nodes, dim, n_edges):
    nw = _NUM_SC * _NUM_TILES
    per_tile = n_edges // nw
    assert per_tile * nw == n_edges and per_tile % _CHUNK == 0
    n_chunks = per_tile // _CHUNK
    rows_per_tile = _pad_rows(n_nodes)
    n_pad = rows_per_tile * _NUM_TILES
    zcopies = rows_per_tile // _ZROWS
    lanes = dim // 16

    mesh = plsc.VectorSubcoreMesh(core_axis_name="c", subcore_axis_name="s")

    _NBUF = 7
    assert n_chunks >= 3 * _NBUF
    _GRP = (n_chunks - _NBUF + 1) // _NBUF

    @functools.partial(
        pl.kernel,
        out_type=jax.ShapeDtypeStruct((_NUM_SC * n_pad, dim), jnp.float32),
        mesh=mesh,
        scratch_types=[
            pltpu.VMEM((per_tile,), jnp.int32),
            [pltpu.VMEM((_CHUNK,), jnp.int32)] * _NBUF,
            [pltpu.VMEM((_CHUNK, dim), jnp.float32)] * _NBUF,
            pltpu.VMEM_SHARED((n_pad, dim), jnp.float32),
            [pltpu.SemaphoreType.DMA] * _NBUF,
            [pltpu.SemaphoreType.DMA] * _NBUF,
            [pltpu.SemaphoreType.DMA] * _NBUF,
            pltpu.SemaphoreType.DMA,
        ],
    )
    def seg_sum(x_hbm, ei_hbm, out_hbm, sidx, didxs, rows,
                acc, gsems, dsems, ssems, isem):
        c = lax.axis_index("c")
        s = lax.axis_index("s")
        wid = c * _NUM_TILES + s
        base0 = wid * per_tile

        pltpu.sync_copy(ei_hbm.at[pl.ds(base0, per_tile)], sidx)

        def start_chunk(i, b):
            pltpu.async_copy(
                ei_hbm.at[pl.ds(n_edges + base0 + i * _CHUNK, _CHUNK)],
                didxs[b], dsems[b])
            pltpu.async_copy(
                x_hbm.at[sidx.at[pl.ds(i * _CHUNK, _CHUNK)]], rows[b],
                gsems[b])

        def wait_chunk(i, b):
            pltpu.make_async_copy(
                x_hbm.at[sidx.at[pl.ds(i * _CHUNK, _CHUNK)]], rows[b],
                gsems[b]).wait()
            pltpu.make_async_copy(
                ei_hbm.at[pl.ds(n_edges + base0 + i * _CHUNK, _CHUNK)],
                didxs[b], dsems[b]).wait()

        def wait_scatter(b):
            pltpu.make_async_copy(rows[b], acc.at[didxs[b]], ssems[b]).wait()

        def stage(i, b, prefetch, wait_prev):
            wait_chunk(i, b)
            pltpu.async_copy(rows[b], acc.at[didxs[b]], ssems[b], add=True)
            nb = (b + _NBUF - 1) % _NBUF
            if wait_prev:
                wait_scatter(nb)
            if prefetch:
                start_chunk(i + _NBUF - 1, nb)

        for j in range(_NBUF - 1):
            start_chunk(j, j)

        def zrow(r, carry):
            for u in range(lanes):
                rows[_NBUF - 1][r, pl.ds(16 * u, 16)] = (
                    jnp.zeros((16,), jnp.float32))
            return carry

        lax.fori_loop(0, _CHUNK, zrow, 0)

        last = n_nodes - (_NUM_TILES - 1) * rows_per_tile
        assert 0 < last <= rows_per_tile and last % 8 == 0
        assert (n_pad - n_nodes) % _CHUNK == 0

        @pl.when(jnp.logical_and(c == 0, s < _NUM_TILES - 1))
        def _():
            pltpu.async_copy(x_hbm.at[pl.ds(s * rows_per_tile, rows_per_tile)],
                             acc.at[pl.ds(s * rows_per_tile, rows_per_tile)],
                             isem).wait()

        @pl.when(jnp.logical_and(c == 0, s == _NUM_TILES - 1))
        def _():
            base = (_NUM_TILES - 1) * rows_per_tile
            npadz = (n_pad - n_nodes) // _CHUNK
            pltpu.async_copy(x_hbm.at[pl.ds(base, last)],
                             acc.at[pl.ds(base, last)], isem)
            for z in range(npadz):
                pltpu.async_copy(
                    rows[_NBUF - 1],
                    acc.at[pl.ds(n_nodes + z * _CHUNK, _CHUNK)], isem)
            pltpu.make_async_copy(x_hbm.at[pl.ds(base, last)],
                                  acc.at[pl.ds(base, last)], isem).wait()
            for z in range(npadz):
                pltpu.make_async_copy(
                    rows[_NBUF - 1],
                    acc.at[pl.ds(n_nodes + z * _CHUNK, _CHUNK)], isem).wait()

        @pl.when(c == 1)
        def _():
            nz = rows_per_tile // _CHUNK
            for z in range(nz):
                pltpu.async_copy(
                    rows[_NBUF - 1],
                    acc.at[pl.ds(s * rows_per_tile + z * _CHUNK, _CHUNK)],
                    isem)
            for z in range(nz):
                pltpu.make_async_copy(
                    rows[_NBUF - 1],
                    acc.at[pl.ds(s * rows_per_tile + z * _CHUNK, _CHUNK)],
                    isem).wait()

        plsc.subcore_barrier()
        stage(0, 0, True, False)
        for j in range(1, _NBUF):
            stage(j, j, True, True)

        def group(p, carry):
            i0 = _NBUF * p
            for u in range(_NBUF):
                stage(i0 + u, u, True, True)
            return carry

        lax.fori_loop(1, _GRP, group, 0)
        for i in range(_NBUF * _GRP, n_chunks):
            stage(i, i % _NBUF, i + _NBUF - 1 < n_chunks, True)
        wait_scatter((n_chunks - 1) % _NBUF)

        plsc.subcore_barrier()
        pltpu.sync_copy(
            acc.at[pl.ds(s * rows_per_tile, rows_per_tile)],
            out_hbm.at[pl.ds(c * n_pad + s * rows_per_tile, rows_per_tile)])

    return seg_sum


def _gin_dense(n, p_ref, w1_ref, b1_ref, g_ref, bt_ref, w2_ref, b2_ref):
    n_pad = p_ref.shape[0] // 2
    h = p_ref[:n, :] + p_ref[n_pad:n_pad + n, :]
    t = (jnp.dot(h, w1_ref[...], preferred_element_type=jnp.float32)
         + b1_ref[...][None, :])
    m = jnp.mean(t, axis=0, keepdims=True)
    d = t - m
    v = jnp.mean(d * d, axis=0, keepdims=True)
    t = (g_ref[...][None, :] * d * lax.rsqrt(v + 1e-5)
         + bt_ref[...][None, :])
    t = jnp.maximum(t, 0.0)
    t = (jnp.dot(t, w2_ref[...], preferred_element_type=jnp.float32)
         + b2_ref[...][None, :])
    return jnp.maximum(t, 0.0)


def _dense_body(p_ref, w1_ref, b1_ref, g_ref, bt_ref, w2_ref, b2_ref,
                o_ref):
    o_ref[...] = _gin_dense(o_ref.shape[0], p_ref, w1_ref, b1_ref, g_ref,
                            bt_ref, w2_ref, b2_ref)


def _final_body(p_ref, batch_ref, w1_ref, b1_ref, g_ref, bt_ref,
                w2_ref, b2_ref, mw1_ref, mb1_ref, mw2_ref, mb2_ref, o_ref):
    n = batch_ref.shape[0]
    h = _gin_dense(n, p_ref, w1_ref, b1_ref, g_ref, bt_ref, w2_ref, b2_ref)
    n_graphs = o_ref.shape[0]
    onehot_t = (lax.broadcasted_iota(jnp.int32, (n_graphs, n), 0)
                == batch_ref[...][None, :]).astype(jnp.float32)
    pooled = jnp.dot(onehot_t, h, preferred_element_type=jnp.float32)
    u = jnp.maximum(
        jnp.dot(pooled, mw1_ref[...], preferred_element_type=jnp.float32)
        + mb1_ref[...][None, :], 0.0)
    o_ref[...] = (jnp.dot(u, mw2_ref[...], preferred_element_type=jnp.float32)
                  + mb2_ref[...][None, :])


def kernel(x, edge_index, batch, batch_size, c0_W1, c0_b1, c0_g, c0_bt, c0_W2,
           c0_b2, c1_W1, c1_b1, c1_g, c1_bt, c1_W2, c1_b2, m_W1, m_b1, m_W2,
           m_b2):
    n, d = x.shape
    e = edge_index.shape[1]
    hid = c0_W1.shape[1]
    out_d = m_W2.shape[1]
    n_graphs = batch_size if isinstance(batch_size, int) else 64

    ei_flat = edge_index.reshape(-1)

    seg_sum = _make_seg_sum(n, d, e)

    parts0 = seg_sum(x, ei_flat)
    h0 = pl.pallas_call(
        _dense_body,
        out_shape=jax.ShapeDtypeStruct((n, hid), jnp.float32),
    )(parts0, c0_W1, c0_b1, c0_g, c0_bt, c0_W2, c0_b2)

    parts1 = seg_sum(h0, ei_flat)
    out = pl.pallas_call(
        _final_body,
        out_shape=jax.ShapeDtypeStruct((n_graphs, out_d), jnp.float32),
    )(parts1, batch, c1_W1, c1_b1, c1_g, c1_bt, c1_W2,
      c1_b2, m_W1, m_b1, m_W2, m_b2)
    return out
